# baseline (speedup 1.0000x reference)
import functools

import jax
import jax.numpy as jnp
from jax import lax
from jax.experimental import pallas as pl
from jax.experimental.pallas import tpu as pltpu
from jax.experimental.pallas import tpu_sc as plsc

NC = 2
NS = 16
NW = NC * NS
CHUNK = 64


def _sc_pair_gather(pairs, pidx, n_chunks, DP):
    mesh = plsc.VectorSubcoreMesh(core_axis_name="c", subcore_axis_name="s")
    N = NW * n_chunks * CHUNK
    per_w = n_chunks * CHUNK

    @functools.partial(
        pl.kernel,
        mesh=mesh,
        compiler_params=pltpu.CompilerParams(use_tc_tiling_on_sc=False),
        out_type=jax.ShapeDtypeStruct((N, DP), jnp.float32),
        scratch_types=[
            pltpu.VMEM((n_chunks, CHUNK), jnp.int32),
            pltpu.VMEM((2, CHUNK, DP), jnp.float32),
            pltpu.SemaphoreType.DMA,
            pltpu.SemaphoreType.DMA,
        ],
    )
    def body(pairs_hbm, pidx_hbm, out_hbm, idx_v, rows_v, gsem, ssem):
        wid = lax.axis_index("s") * NC + lax.axis_index("c")
        base = wid * per_w
        pltpu.sync_copy(pidx_hbm.at[wid], idx_v)

        def gather_start(g, buf):
            pltpu.async_copy(pairs_hbm.at[idx_v.at[g]], rows_v.at[buf], gsem)

        def gather_wait(g, buf):
            pltpu.make_async_copy(
                pairs_hbm.at[idx_v.at[g]], rows_v.at[buf], gsem).wait()

        def store_start(g):
            pltpu.async_copy(
                rows_v.at[lax.rem(g, 2)],
                out_hbm.at[pl.ds(base + g * CHUNK, CHUNK)], ssem)

        def store_wait(g):
            pltpu.make_async_copy(
                rows_v.at[lax.rem(g, 2)],
                out_hbm.at[pl.ds(base + g * CHUNK, CHUNK)], ssem).wait()

        gather_start(0, 0)

        def step(g, carry):
            gather_wait(g, lax.rem(g, 2))
            store_start(g)

            @pl.when(g >= 1)
            def _():
                store_wait(g - 1)

            @pl.when(g <= n_chunks - 2)
            def _():
                gather_start(g + 1, lax.rem(g + 1, 2))

            return carry

        lax.fori_loop(0, n_chunks, step, 0)
        store_wait(n_chunks - 1)

    return body(pairs, pidx)


def _tc_select_half(pairs_out, parity, D):
    N = pairs_out.shape[0]
    BLK = 512

    def body(pairs_ref, par_ref, out_ref):
        left = pairs_ref[:, :D]
        right = pairs_ref[:, D:]
        sel = par_ref[...] != 0
        out_ref[...] = jnp.where(sel, right, left)

    return pl.pallas_call(
        body,
        grid=(N // BLK,),
        in_specs=[
            pl.BlockSpec((BLK, 2 * D), lambda i: (i, 0)),
            pl.BlockSpec((BLK, 1), lambda i: (i, 0)),
        ],
        out_specs=pl.BlockSpec((BLK, D), lambda i: (i, 0)),
        out_shape=jax.ShapeDtypeStruct((N, D), jnp.float32),
    )(pairs_out, parity)


def kernel(table, indices):
    V, D = table.shape
    B, H = indices.shape
    N = B * H
    assert V % 2 == 0 and N % (NW * CHUNK) == 0
    n_chunks = N // (NW * CHUNK)
    idx = indices.astype(jnp.int32).reshape(N)
    pairs = table.reshape(V // 2, 2 * D)
    pidx = (idx >> 1).reshape(NW, n_chunks, CHUNK)
    parity = (idx & 1).reshape(N, 1)
    pairs_out = _sc_pair_gather(pairs, pidx, n_chunks, 2 * D)
    out = _tc_select_half(pairs_out, parity, D)
    return out.reshape(B, H, D)


# speedup vs baseline: 2.8141x; 2.8141x over previous
import functools

import jax
import jax.numpy as jnp
from jax import lax
from jax.experimental import pallas as pl
from jax.experimental.pallas import tpu as pltpu
from jax.experimental.pallas import tpu_sc as plsc

NC = 2
NS = 16
NW = NC * NS
CHUNK = 128
LB = 128


def _tc_tails(table, V, D, DT):
    R = 1000
    TAIL = D - 2 * LB

    def body(t_ref, o_ref):
        lane = lax.broadcasted_iota(jnp.int32, (R, LB), 1)
        o_ref[...] = jnp.where(lane < TAIL, t_ref[...], 0.0)

    return pl.pallas_call(
        body,
        grid=(V // R,),
        in_specs=[pl.BlockSpec((R, LB), lambda i: (i, 2))],
        out_specs=pl.BlockSpec((R, LB), lambda i: (i, 0)),
        out_shape=jax.ShapeDtypeStruct((V, DT), jnp.float32),
    )(table)


def _sc_gather(table, tails, idx, n_chunks, D, DP):
    mesh = plsc.VectorSubcoreMesh(core_axis_name="c", subcore_axis_name="s")
    N = NW * n_chunks * CHUNK
    per_w = n_chunks * CHUNK

    @functools.partial(
        pl.kernel,
        mesh=mesh,
        compiler_params=pltpu.CompilerParams(use_tc_tiling_on_sc=True),
        out_type=jax.ShapeDtypeStruct((N, DP), jnp.float32),
        scratch_types=[
            pltpu.VMEM((per_w,), jnp.int32),
            pltpu.VMEM((2, CHUNK, LB), jnp.float32),
            pltpu.VMEM((2, CHUNK, LB), jnp.float32),
            pltpu.VMEM((2, CHUNK, LB), jnp.float32),
            pltpu.SemaphoreType.DMA,
            pltpu.SemaphoreType.DMA,
        ],
    )
    def body(t_hbm, tl_hbm, i_hbm, o_hbm, i_v, a_v, b_v, c_v, gsem, ssem):
        wid = lax.axis_index("s") * NC + lax.axis_index("c")
        base = wid * per_w
        pltpu.sync_copy(i_hbm.at[pl.ds(base, per_w)], i_v)

        def gather_start(g, buf):
            iv = i_v.at[pl.ds(g * CHUNK, CHUNK)]
            pltpu.async_copy(t_hbm.at[iv, pl.ds(0, LB)], a_v.at[buf], gsem)
            pltpu.async_copy(t_hbm.at[iv, pl.ds(LB, LB)], b_v.at[buf], gsem)
            pltpu.async_copy(tl_hbm.at[iv], c_v.at[buf], gsem)

        def gather_wait(g, buf):
            iv = i_v.at[pl.ds(g * CHUNK, CHUNK)]
            pltpu.make_async_copy(
                t_hbm.at[iv, pl.ds(0, LB)], a_v.at[buf], gsem).wait()
            pltpu.make_async_copy(
                t_hbm.at[iv, pl.ds(LB, LB)], b_v.at[buf], gsem).wait()
            pltpu.make_async_copy(tl_hbm.at[iv], c_v.at[buf], gsem).wait()

        def rows_of(g):
            return pl.ds(base + g * CHUNK, CHUNK)

        def store_start(g):
            buf = lax.rem(g, 2)
            pltpu.async_copy(a_v.at[buf], o_hbm.at[rows_of(g), pl.ds(0, LB)],
                             ssem)
            pltpu.async_copy(b_v.at[buf], o_hbm.at[rows_of(g), pl.ds(LB, LB)],
                             ssem)
            pltpu.async_copy(c_v.at[buf],
                             o_hbm.at[rows_of(g), pl.ds(2 * LB, LB)], ssem)

        def store_wait(g):
            buf = lax.rem(g, 2)
            pltpu.make_async_copy(
                a_v.at[buf], o_hbm.at[rows_of(g), pl.ds(0, LB)], ssem).wait()
            pltpu.make_async_copy(
                b_v.at[buf], o_hbm.at[rows_of(g), pl.ds(LB, LB)], ssem).wait()
            pltpu.make_async_copy(
                c_v.at[buf], o_hbm.at[rows_of(g), pl.ds(2 * LB, LB)],
                ssem).wait()

        gather_start(0, 0)

        def step(g, carry):
            gather_wait(g, lax.rem(g, 2))
            store_start(g)

            @pl.when(g >= 1)
            def _():
                store_wait(g - 1)

            @pl.when(g <= n_chunks - 2)
            def _():
                gather_start(g + 1, lax.rem(g + 1, 2))

            return carry

        lax.fori_loop(0, n_chunks, step, 0)
        store_wait(n_chunks - 1)

    return body(table, tails, idx)


def _tc_unpad(g384, N, D, DP):
    BLK = 512

    def body(i_ref, o_ref):
        o_ref[...] = i_ref[:, :D]

    return pl.pallas_call(
        body,
        grid=(N // BLK,),
        in_specs=[pl.BlockSpec((BLK, DP), lambda i: (i, 0))],
        out_specs=pl.BlockSpec((BLK, D), lambda i: (i, 0)),
        out_shape=jax.ShapeDtypeStruct((N, D), jnp.float32),
    )(g384)


def kernel(table, indices):
    V, D = table.shape
    B, H = indices.shape
    N = B * H
    DP = 3 * LB
    DT = LB
    assert N % (NW * CHUNK) == 0
    n_chunks = N // (NW * CHUNK)
    idx = indices.astype(jnp.int32).reshape(N)
    tails = _tc_tails(table, V, D, DT)
    g384 = _sc_gather(table, tails, idx, n_chunks, D, DP)
    out = _tc_unpad(g384, N, D, DP)
    return out.reshape(B, H, D)


# speedup vs baseline: 3.3396x; 1.1867x over previous
import functools

import jax
import jax.numpy as jnp
from jax import lax
from jax.experimental import pallas as pl
from jax.experimental.pallas import tpu as pltpu
from jax.experimental.pallas import tpu_sc as plsc

NC = 2
NS = 16
NW = NC * NS
CHUNK = 128
LB = 128
VB = 512


def _tc_repack(tableT, V, D, DP):
    def body(t_ref, o_ref):
        xt = jnp.swapaxes(t_ref[...], 0, 1)
        lane = lax.broadcasted_iota(jnp.int32, (VB, DP), 1)
        o_ref[...] = jnp.where(lane < D, jnp.pad(xt, ((0, 0), (0, DP - D))),
                               0.0)

    return pl.pallas_call(
        body,
        grid=(pl.cdiv(V, VB),),
        in_specs=[pl.BlockSpec((D, VB), lambda j: (0, j))],
        out_specs=pl.BlockSpec((VB, DP), lambda j: (j, 0)),
        out_shape=jax.ShapeDtypeStruct((V, DP), jnp.float32),
    )(tableT)


def _sc_gather(tbl384, idx, n_chunks, DP):
    mesh = plsc.VectorSubcoreMesh(core_axis_name="c", subcore_axis_name="s")
    N = NW * n_chunks * CHUNK
    per_w = n_chunks * CHUNK

    @functools.partial(
        pl.kernel,
        mesh=mesh,
        compiler_params=pltpu.CompilerParams(use_tc_tiling_on_sc=True),
        out_type=jax.ShapeDtypeStruct((N, DP), jnp.float32),
        scratch_types=[
            pltpu.VMEM((per_w,), jnp.int32),
            pltpu.VMEM((2, CHUNK, DP), jnp.float32),
            pltpu.SemaphoreType.DMA,
            pltpu.SemaphoreType.DMA,
        ],
    )
    def body(t_hbm, i_hbm, o_hbm, i_v, r_v, gsem, ssem):
        wid = lax.axis_index("s") * NC + lax.axis_index("c")
        base = wid * per_w
        pltpu.sync_copy(i_hbm.at[pl.ds(base, per_w)], i_v)

        def gather_start(g, buf):
            iv = i_v.at[pl.ds(g * CHUNK, CHUNK)]
            pltpu.async_copy(t_hbm.at[iv], r_v.at[buf], gsem)

        def gather_wait(g, buf):
            iv = i_v.at[pl.ds(g * CHUNK, CHUNK)]
            pltpu.make_async_copy(t_hbm.at[iv], r_v.at[buf], gsem).wait()

        def store_start(g):
            pltpu.async_copy(
                r_v.at[lax.rem(g, 2)],
                o_hbm.at[pl.ds(base + g * CHUNK, CHUNK)], ssem)

        def store_wait(g):
            pltpu.make_async_copy(
                r_v.at[lax.rem(g, 2)],
                o_hbm.at[pl.ds(base + g * CHUNK, CHUNK)], ssem).wait()

        gather_start(0, 0)

        def step(g, carry):
            gather_wait(g, lax.rem(g, 2))
            store_start(g)

            @pl.when(g >= 1)
            def _():
                store_wait(g - 1)

            @pl.when(g <= n_chunks - 2)
            def _():
                gather_start(g + 1, lax.rem(g + 1, 2))

            return carry

        lax.fori_loop(0, n_chunks, step, 0)
        store_wait(n_chunks - 1)

    return body(tbl384, idx)


def _tc_unpad(g384, B, H, D, DP):
    BB = 16

    def body(i_ref, o_ref):
        o_ref[...] = i_ref[:, :D].reshape(BB, H, D)

    return pl.pallas_call(
        body,
        grid=(B // BB,),
        in_specs=[pl.BlockSpec((BB * H, DP), lambda i: (i, 0))],
        out_specs=pl.BlockSpec((BB, H, D), lambda i: (i, 0, 0)),
        out_shape=jax.ShapeDtypeStruct((B, H, D), jnp.float32),
    )(g384)


def kernel(table, indices):
    V, D = table.shape
    B, H = indices.shape
    N = B * H
    DP = 3 * LB
    assert N % (NW * CHUNK) == 0
    n_chunks = N // (NW * CHUNK)
    idx = indices.astype(jnp.int32).reshape(N)
    tbl384 = _tc_repack(table.T, V, D, DP)
    g384 = _sc_gather(tbl384, idx, n_chunks, DP)
    return _tc_unpad(g384, B, H, D, DP)


# speedup vs baseline: 4.1318x; 1.2372x over previous
import functools

import jax
import jax.numpy as jnp
from jax import lax
from jax.experimental import pallas as pl
from jax.experimental.pallas import tpu as pltpu
from jax.experimental.pallas import tpu_sc as plsc

NC = 2
NS = 16
NW = NC * NS
CHUNK = 128
LB = 128
VB = 1024


def _tc_repack(tableT, V, D, DP):
    def body(t_ref, o_ref):
        o_ref[:, :D] = jnp.swapaxes(t_ref[...], 0, 1)
        o_ref[:, D:] = jnp.zeros((VB, DP - D), jnp.float32)

    return pl.pallas_call(
        body,
        grid=(pl.cdiv(V, VB),),
        in_specs=[pl.BlockSpec((D, VB), lambda j: (0, j))],
        out_specs=pl.BlockSpec((VB, DP), lambda j: (j, 0)),
        out_shape=jax.ShapeDtypeStruct((V, DP), jnp.float32),
    )(tableT)


def _sc_gather(tbl384, idx, n_chunks, DP):
    mesh = plsc.VectorSubcoreMesh(core_axis_name="c", subcore_axis_name="s")
    N = NW * n_chunks * CHUNK
    per_w = n_chunks * CHUNK

    @functools.partial(
        pl.kernel,
        mesh=mesh,
        compiler_params=pltpu.CompilerParams(use_tc_tiling_on_sc=True),
        out_type=jax.ShapeDtypeStruct((N, DP), jnp.float32),
        scratch_types=[
            pltpu.VMEM((per_w,), jnp.int32),
            pltpu.VMEM((2, CHUNK, DP), jnp.float32),
            pltpu.SemaphoreType.DMA,
            pltpu.SemaphoreType.DMA,
        ],
    )
    def body(t_hbm, i_hbm, o_hbm, i_v, r_v, gsem, ssem):
        wid = lax.axis_index("s") * NC + lax.axis_index("c")
        base = wid * per_w
        pltpu.sync_copy(i_hbm.at[pl.ds(base, per_w)], i_v)

        def gather_start(g, buf):
            iv = i_v.at[pl.ds(g * CHUNK, CHUNK)]
            pltpu.async_copy(t_hbm.at[iv], r_v.at[buf], gsem)

        def gather_wait(g, buf):
            iv = i_v.at[pl.ds(g * CHUNK, CHUNK)]
            pltpu.make_async_copy(t_hbm.at[iv], r_v.at[buf], gsem).wait()

        def store_start(g):
            pltpu.async_copy(
                r_v.at[lax.rem(g, 2)],
                o_hbm.at[pl.ds(base + g * CHUNK, CHUNK)], ssem)

        def store_wait(g):
            pltpu.make_async_copy(
                r_v.at[lax.rem(g, 2)],
                o_hbm.at[pl.ds(base + g * CHUNK, CHUNK)], ssem).wait()

        gather_start(0, 0)

        def step(g, carry):
            gather_wait(g, lax.rem(g, 2))
            store_start(g)

            @pl.when(g >= 1)
            def _():
                store_wait(g - 1)

            @pl.when(g <= n_chunks - 2)
            def _():
                gather_start(g + 1, lax.rem(g + 1, 2))

            return carry

        lax.fori_loop(0, n_chunks, step, 0)
        store_wait(n_chunks - 1)

    return body(tbl384, idx)


def _tc_unpad(g384, B, H, D, DP):
    BB = 16

    def body(i_ref, o_ref):
        o_ref[...] = i_ref[:, :D].reshape(BB, H, D)

    return pl.pallas_call(
        body,
        grid=(B // BB,),
        in_specs=[pl.BlockSpec((BB * H, DP), lambda i: (i, 0))],
        out_specs=pl.BlockSpec((BB, H, D), lambda i: (i, 0, 0)),
        out_shape=jax.ShapeDtypeStruct((B, H, D), jnp.float32),
    )(g384)


def kernel(table, indices):
    V, D = table.shape
    B, H = indices.shape
    N = B * H
    DP = 3 * LB
    assert N % (NW * CHUNK) == 0
    n_chunks = N // (NW * CHUNK)
    idx = indices.astype(jnp.int32).reshape(N)
    tbl384 = _tc_repack(table.T, V, D, DP)
    g384 = _sc_gather(tbl384, idx, n_chunks, DP)
    return _tc_unpad(g384, B, H, D, DP)


# speedup vs baseline: 4.8344x; 1.1701x over previous
import functools

import jax
import jax.numpy as jnp
from jax import lax
from jax.experimental import pallas as pl
from jax.experimental.pallas import tpu as pltpu
from jax.experimental.pallas import tpu_sc as plsc

NC = 2
NS = 16
NW = NC * NS
CHUNK = 128
LB = 128
VB = 2048


def _tc_repack(tableT, V, D, DP):
    def body(t_ref, o_ref):
        o_ref[:, :D] = jnp.swapaxes(t_ref[...], 0, 1)
        o_ref[:, D:] = jnp.zeros((VB, DP - D), jnp.float32)

    return pl.pallas_call(
        body,
        grid=(pl.cdiv(V, VB),),
        in_specs=[pl.BlockSpec((D, VB), lambda j: (0, j))],
        out_specs=pl.BlockSpec((VB, DP), lambda j: (j, 0)),
        out_shape=jax.ShapeDtypeStruct((V, DP), jnp.float32),
    )(tableT)


def _sc_gather(tbl384, idx, n_chunks, DP):
    mesh = plsc.VectorSubcoreMesh(core_axis_name="c", subcore_axis_name="s")
    N = NW * n_chunks * CHUNK
    per_w = n_chunks * CHUNK

    @functools.partial(
        pl.kernel,
        mesh=mesh,
        compiler_params=pltpu.CompilerParams(use_tc_tiling_on_sc=True),
        out_type=jax.ShapeDtypeStruct((N, DP), jnp.float32),
        scratch_types=[
            pltpu.VMEM((per_w,), jnp.int32),
            pltpu.VMEM((2, CHUNK, DP), jnp.float32),
            pltpu.SemaphoreType.DMA,
            pltpu.SemaphoreType.DMA,
        ],
    )
    def body(t_hbm, i_hbm, o_hbm, i_v, r_v, gsem, ssem):
        wid = lax.axis_index("s") * NC + lax.axis_index("c")
        base = wid * per_w
        pltpu.sync_copy(i_hbm.at[pl.ds(base, per_w)], i_v)

        def gather_start(g, buf):
            iv = i_v.at[pl.ds(g * CHUNK, CHUNK)]
            pltpu.async_copy(t_hbm.at[iv], r_v.at[buf], gsem)

        def gather_wait(g, buf):
            iv = i_v.at[pl.ds(g * CHUNK, CHUNK)]
            pltpu.make_async_copy(t_hbm.at[iv], r_v.at[buf], gsem).wait()

        def store_start(g):
            pltpu.async_copy(
                r_v.at[lax.rem(g, 2)],
                o_hbm.at[pl.ds(base + g * CHUNK, CHUNK)], ssem)

        def store_wait(g):
            pltpu.make_async_copy(
                r_v.at[lax.rem(g, 2)],
                o_hbm.at[pl.ds(base + g * CHUNK, CHUNK)], ssem).wait()

        gather_start(0, 0)

        def step(g, carry):
            gather_wait(g, lax.rem(g, 2))
            store_start(g)

            @pl.when(g >= 1)
            def _():
                store_wait(g - 1)

            @pl.when(g <= n_chunks - 2)
            def _():
                gather_start(g + 1, lax.rem(g + 1, 2))

            return carry

        lax.fori_loop(0, n_chunks, step, 0)
        store_wait(n_chunks - 1)

    return body(tbl384, idx)


def _tc_unpad(g384, B, H, D, DP):
    BB = 16

    def body(i_ref, o_ref):
        o_ref[...] = i_ref[:, :D].reshape(BB, H, D)

    return pl.pallas_call(
        body,
        grid=(B // BB,),
        in_specs=[pl.BlockSpec((BB * H, DP), lambda i: (i, 0))],
        out_specs=pl.BlockSpec((BB, H, D), lambda i: (i, 0, 0)),
        out_shape=jax.ShapeDtypeStruct((B, H, D), jnp.float32),
    )(g384)


def kernel(table, indices):
    V, D = table.shape
    B, H = indices.shape
    N = B * H
    DP = 3 * LB
    assert N % (NW * CHUNK) == 0
    n_chunks = N // (NW * CHUNK)
    idx = indices.astype(jnp.int32).reshape(N)
    tbl384 = _tc_repack(table.T, V, D, DP)
    g384 = _sc_gather(tbl384, idx, n_chunks, DP)
    return _tc_unpad(g384, B, H, D, DP)


# speedup vs baseline: 5.0404x; 1.0426x over previous
import functools

import jax
import jax.numpy as jnp
from jax import lax
from jax.experimental import pallas as pl
from jax.experimental.pallas import tpu as pltpu
from jax.experimental.pallas import tpu_sc as plsc

NC = 2
NS = 16
NW = NC * NS
CHUNK = 128
LB = 128
VB = 4096


def _tc_repack(tableT, V, D, DP):
    def body(t_ref, o_ref):
        o_ref[:, :D] = jnp.swapaxes(t_ref[...], 0, 1)
        o_ref[:, D:] = jnp.zeros((VB, DP - D), jnp.float32)

    return pl.pallas_call(
        body,
        grid=(pl.cdiv(V, VB),),
        in_specs=[pl.BlockSpec((D, VB), lambda j: (0, j))],
        out_specs=pl.BlockSpec((VB, DP), lambda j: (j, 0)),
        out_shape=jax.ShapeDtypeStruct((V, DP), jnp.float32),
    )(tableT)


def _sc_gather(tbl384, idx, n_chunks, DP):
    mesh = plsc.VectorSubcoreMesh(core_axis_name="c", subcore_axis_name="s")
    N = NW * n_chunks * CHUNK
    per_w = n_chunks * CHUNK

    @functools.partial(
        pl.kernel,
        mesh=mesh,
        compiler_params=pltpu.CompilerParams(use_tc_tiling_on_sc=True),
        out_type=jax.ShapeDtypeStruct((N, DP), jnp.float32),
        scratch_types=[
            pltpu.VMEM((per_w,), jnp.int32),
            pltpu.VMEM((2, CHUNK, DP), jnp.float32),
            pltpu.SemaphoreType.DMA,
            pltpu.SemaphoreType.DMA,
        ],
    )
    def body(t_hbm, i_hbm, o_hbm, i_v, r_v, gsem, ssem):
        wid = lax.axis_index("s") * NC + lax.axis_index("c")
        base = wid * per_w
        pltpu.sync_copy(i_hbm.at[pl.ds(base, per_w)], i_v)

        def gather_start(g, buf):
            iv = i_v.at[pl.ds(g * CHUNK, CHUNK)]
            pltpu.async_copy(t_hbm.at[iv], r_v.at[buf], gsem)

        def gather_wait(g, buf):
            iv = i_v.at[pl.ds(g * CHUNK, CHUNK)]
            pltpu.make_async_copy(t_hbm.at[iv], r_v.at[buf], gsem).wait()

        def store_start(g):
            pltpu.async_copy(
                r_v.at[lax.rem(g, 2)],
                o_hbm.at[pl.ds(base + g * CHUNK, CHUNK)], ssem)

        def store_wait(g):
            pltpu.make_async_copy(
                r_v.at[lax.rem(g, 2)],
                o_hbm.at[pl.ds(base + g * CHUNK, CHUNK)], ssem).wait()

        gather_start(0, 0)

        def step(g, carry):
            gather_wait(g, lax.rem(g, 2))
            store_start(g)

            @pl.when(g >= 1)
            def _():
                store_wait(g - 1)

            @pl.when(g <= n_chunks - 2)
            def _():
                gather_start(g + 1, lax.rem(g + 1, 2))

            return carry

        lax.fori_loop(0, n_chunks, step, 0)
        store_wait(n_chunks - 1)

    return body(tbl384, idx)


def _tc_unpad(g384, B, H, D, DP):
    BB = 16

    def body(i_ref, o_ref):
        o_ref[...] = i_ref[:, :D].reshape(BB, H, D)

    return pl.pallas_call(
        body,
        grid=(B // BB,),
        in_specs=[pl.BlockSpec((BB * H, DP), lambda i: (i, 0))],
        out_specs=pl.BlockSpec((BB, H, D), lambda i: (i, 0, 0)),
        out_shape=jax.ShapeDtypeStruct((B, H, D), jnp.float32),
    )(g384)


def kernel(table, indices):
    V, D = table.shape
    B, H = indices.shape
    N = B * H
    DP = 3 * LB
    assert N % (NW * CHUNK) == 0
    n_chunks = N // (NW * CHUNK)
    idx = indices.astype(jnp.int32).reshape(N)
    tbl384 = _tc_repack(table.T, V, D, DP)
    g384 = _sc_gather(tbl384, idx, n_chunks, DP)
    return _tc_unpad(g384, B, H, D, DP)


# speedup vs baseline: 5.3303x; 1.0575x over previous
import functools

import jax
import jax.numpy as jnp
from jax import lax
from jax.experimental import pallas as pl
from jax.experimental.pallas import tpu as pltpu
from jax.experimental.pallas import tpu_sc as plsc

NC = 2
NS = 16
NW = NC * NS
CHUNK = 128
LB = 128
VB = 4096


def _tc_repack(tableT, V, D, DP):
    def body(t_ref, o_ref):
        o_ref[:, :D] = jnp.swapaxes(t_ref[...], 0, 1)
        o_ref[:, D:] = jnp.zeros((VB, DP - D), jnp.float32)

    return pl.pallas_call(
        body,
        grid=(pl.cdiv(V, VB),),
        in_specs=[pl.BlockSpec((D, VB), lambda j: (0, j))],
        out_specs=pl.BlockSpec((VB, DP), lambda j: (j, 0)),
        out_shape=jax.ShapeDtypeStruct((V, DP), jnp.float32),
    )(tableT)


def _sc_gather(tbl384, idx, n_chunks, DP):
    mesh = plsc.VectorSubcoreMesh(core_axis_name="c", subcore_axis_name="s")
    N = NW * n_chunks * CHUNK
    per_w = n_chunks * CHUNK

    @functools.partial(
        pl.kernel,
        mesh=mesh,
        compiler_params=pltpu.CompilerParams(use_tc_tiling_on_sc=True),
        out_type=jax.ShapeDtypeStruct((N, DP), jnp.float32),
        scratch_types=[
            pltpu.VMEM((per_w,), jnp.int32),
            pltpu.VMEM((2, CHUNK, DP), jnp.float32),
            pltpu.SemaphoreType.DMA,
            pltpu.SemaphoreType.DMA,
        ],
    )
    def body(t_hbm, i_hbm, o_hbm, i_v, r_v, gsem, ssem):
        wid = lax.axis_index("s") * NC + lax.axis_index("c")
        base = wid * per_w
        pltpu.sync_copy(i_hbm.at[pl.ds(base, per_w)], i_v)

        def gather_start(g, buf):
            iv = i_v.at[pl.ds(g * CHUNK, CHUNK)]
            pltpu.async_copy(t_hbm.at[iv], r_v.at[buf], gsem)

        def gather_wait(g, buf):
            iv = i_v.at[pl.ds(g * CHUNK, CHUNK)]
            pltpu.make_async_copy(t_hbm.at[iv], r_v.at[buf], gsem).wait()

        def store_start(g):
            pltpu.async_copy(
                r_v.at[lax.rem(g, 2)],
                o_hbm.at[pl.ds(base + g * CHUNK, CHUNK)], ssem)

        def store_wait(g):
            pltpu.make_async_copy(
                r_v.at[lax.rem(g, 2)],
                o_hbm.at[pl.ds(base + g * CHUNK, CHUNK)], ssem).wait()

        gather_start(0, 0)

        def step(g, carry):
            gather_wait(g, lax.rem(g, 2))
            store_start(g)

            @pl.when(g >= 1)
            def _():
                store_wait(g - 1)

            @pl.when(g <= n_chunks - 2)
            def _():
                gather_start(g + 1, lax.rem(g + 1, 2))

            return carry

        lax.fori_loop(0, n_chunks, step, 0)
        store_wait(n_chunks - 1)

    return body(tbl384, idx)


def _tc_unpad(g384, B, H, D, DP):
    BB = 16

    def body(i_ref, o_ref):
        o_ref[...] = i_ref[:, :D].reshape(BB, H, D)

    return pl.pallas_call(
        body,
        grid=(B // BB,),
        in_specs=[pl.BlockSpec((BB * H, DP), lambda i: (i, 0))],
        out_specs=pl.BlockSpec((BB, H, D), lambda i: (i, 0, 0)),
        out_shape=jax.ShapeDtypeStruct((B, H, D), jnp.float32),
    )(g384)


def kernel(table, indices):
    V, D = table.shape
    B, H = indices.shape
    N = B * H
    DP = 3 * LB
    assert N % (NW * CHUNK) == 0
    n_chunks = N // (NW * CHUNK)
    idx = indices.astype(jnp.int32).reshape(N)
    tbl384 = _tc_repack(table.T, V, D, DP)
    g384 = _sc_gather(tbl384, idx, n_chunks, DP)
    return g384[:, :D].reshape(B, H, D)
